# Initial kernel scaffold; baseline (speedup 1.0000x reference)
import jax
import jax.numpy as jnp
from jax.experimental import pallas as pl


def kernel(x, edge_index, W1, b1, W2, b2):
    raise NotImplementedError("write your pallas kernel here")


# speedup vs baseline: 18.2970x; 18.2970x over previous
import functools

import jax
import jax.numpy as jnp
from jax import lax
from jax.experimental import pallas as pl
from jax.experimental.pallas import tpu as pltpu
from jax.experimental.pallas import tpu_sc as plsc

N_NODES = 10000
N_EDGES = 320000
D_FEAT = 128
HIDDEN = 128
N_CLASSES = 64

NC = 2
NS = 16
NW = NC * NS
EPT = N_EDGES // NW
CHUNK = 80
NCH = EPT // CHUNK
N_PAD = 10240
RPT = N_PAD // NS
ZROWS = 128

BLK = 1000
GRID = N_NODES // BLK

_mesh = lambda: plsc.VectorSubcoreMesh(core_axis_name="c", subcore_axis_name="s")
_SC_PARAMS = pltpu.CompilerParams(needs_layout_passes=False,
                                  use_tc_tiling_on_sc=False)


@functools.partial(
    pl.kernel,
    out_type=jax.ShapeDtypeStruct((NW, 1, N_NODES), jnp.float32),
    mesh=_mesh(),
    compiler_params=_SC_PARAMS,
    scratch_types=[
        pltpu.VMEM((NCH, CHUNK), jnp.int32),
        pltpu.VMEM((N_NODES,), jnp.float32),
    ],
)
def _degree_kernel(dst_hbm, out_hbm, idx_v, hist_v):
    c = lax.axis_index("c")
    s = lax.axis_index("s")
    wid = s * NC + c
    pltpu.sync_copy(dst_hbm.at[wid], idx_v)
    zeros = jnp.zeros((16,), jnp.float32)

    def zbody(i, carry):
        hist_v[pl.ds(i * 16, 16)] = zeros
        return carry

    lax.fori_loop(0, N_NODES // 16, zbody, 0)
    ones = jnp.ones((16,), jnp.float32)

    def body(j, carry):
        for l in range(CHUNK // 16):
            idx16 = idx_v[j, pl.ds(l * 16, 16)]
            plsc.addupdate_scatter(hist_v, [idx16], ones)
        return carry

    lax.fori_loop(0, NCH, body, 0)
    pltpu.sync_copy(hist_v, out_hbm.at[wid, 0])


def _make_propagate(mode):
    nch = N_EDGES // NS // CHUNK if mode == "cols" else NCH

    @functools.partial(
        pl.kernel,
        out_type=jax.ShapeDtypeStruct((NC, N_PAD, 64), jnp.float32),
        mesh=_mesh(),
        compiler_params=_SC_PARAMS,
        scratch_types=[
            pltpu.VMEM((nch, CHUNK), jnp.int32),
            pltpu.VMEM((nch, CHUNK), jnp.int32),
            pltpu.VMEM((CHUNK, 64), jnp.float32),
            pltpu.VMEM((ZROWS, 64), jnp.float32),
            pltpu.VMEM_SHARED((N_PAD, 64), jnp.float32),
            pltpu.SemaphoreType.DMA,
        ],
    )
    def prop(y_hbm, src_hbm, dst_hbm, out_hbm, src_v, dst_v, rows_v, zbuf, acc, sem):
        c = lax.axis_index("c")
        s = lax.axis_index("s")
        if mode == "cols":
            widx = s
            ysrc = y_hbm.at[c]
        else:
            widx = s * NC + c
            ysrc = y_hbm
        pltpu.sync_copy(src_hbm.at[widx], src_v)
        pltpu.sync_copy(dst_hbm.at[widx], dst_v)

        zeros = jnp.zeros((16,), jnp.float32)

        def zbody(i, carry):
            for l in range(4):
                zbuf[i, pl.ds(l * 16, 16)] = zeros
            return carry

        lax.fori_loop(0, ZROWS, zbody, 0)
        for k in range(RPT // ZROWS):
            pltpu.sync_copy(zbuf, acc.at[pl.ds(s * RPT + k * ZROWS, ZROWS)])
        plsc.subcore_barrier()

        def body(j, carry):
            pltpu.async_copy(ysrc.at[src_v.at[j]], rows_v, sem).wait()
            pltpu.sync_copy(rows_v, acc.at[dst_v.at[j]], add=True)
            return carry

        lax.fori_loop(0, nch, body, 0)
        plsc.subcore_barrier()
        pltpu.sync_copy(acc.at[pl.ds(s * RPT, RPT)],
                        out_hbm.at[c].at[pl.ds(s * RPT, RPT)])

    return prop


_prop_cols = _make_propagate("cols")
_prop_edges = _make_propagate("edges")


def _mm1_body(x_ref, w_ref, o_ref):
    o_ref[...] = jnp.dot(x_ref[...], w_ref[...],
                         preferred_element_type=jnp.float32)


def _mm1(x, w):
    return pl.pallas_call(
        _mm1_body,
        grid=(GRID,),
        in_specs=[
            pl.BlockSpec((BLK, D_FEAT), lambda i: (i, 0)),
            pl.BlockSpec((D_FEAT, HIDDEN), lambda i: (0, 0)),
        ],
        out_specs=pl.BlockSpec((BLK, HIDDEN), lambda i: (i, 0)),
        out_shape=jax.ShapeDtypeStruct((N_NODES, HIDDEN), jnp.float32),
    )(x, w)


def _dinv_body(hist_ref, dinv_ref):
    ones = jnp.ones((NW, 1), jnp.float32)
    deg = lax.dot_general(hist_ref[...], ones, (((0,), (0,)), ((), ())),
                          preferred_element_type=jnp.float32) + 1.0
    dinv_ref[...] = lax.rsqrt(deg)


def _dinv(hist):
    return pl.pallas_call(
        _dinv_body,
        out_shape=jax.ShapeDtypeStruct((N_NODES, 1), jnp.float32),
    )(hist)


def _scale_body(xw_ref, dinv_ref, y_ref):
    y = xw_ref[...] * dinv_ref[...]
    y_ref[0, ...] = y[:, :64]
    y_ref[1, ...] = y[:, 64:]


def _scale(xw, dinv):
    return pl.pallas_call(
        _scale_body,
        grid=(GRID,),
        in_specs=[
            pl.BlockSpec((BLK, HIDDEN), lambda i: (i, 0)),
            pl.BlockSpec((BLK, 1), lambda i: (i, 0)),
        ],
        out_specs=pl.BlockSpec((NC, BLK, 64), lambda i: (0, i, 0)),
        out_shape=jax.ShapeDtypeStruct((NC, N_NODES, 64), jnp.float32),
    )(xw, dinv)


def _mid_body(acc_ref, y1_ref, dinv_ref, b1_ref, w2_ref, y2_ref):
    dinv = dinv_ref[...]
    prop1 = jnp.concatenate([acc_ref[0] + y1_ref[0], acc_ref[1] + y1_ref[1]],
                            axis=1)
    h = jnp.maximum(dinv * prop1 + b1_ref[...], 0.0)
    y2_ref[...] = jnp.dot(h, w2_ref[...],
                          preferred_element_type=jnp.float32) * dinv


def _mid(acc1, y1, dinv, b1, w2):
    return pl.pallas_call(
        _mid_body,
        grid=(GRID,),
        in_specs=[
            pl.BlockSpec((NC, BLK, 64), lambda i: (0, i, 0)),
            pl.BlockSpec((NC, BLK, 64), lambda i: (0, i, 0)),
            pl.BlockSpec((BLK, 1), lambda i: (i, 0)),
            pl.BlockSpec((1, HIDDEN), lambda i: (0, 0)),
            pl.BlockSpec((HIDDEN, N_CLASSES), lambda i: (0, 0)),
        ],
        out_specs=pl.BlockSpec((BLK, N_CLASSES), lambda i: (i, 0)),
        out_shape=jax.ShapeDtypeStruct((N_NODES, N_CLASSES), jnp.float32),
    )(acc1, y1, dinv, b1, w2)


def _final_body(acc_ref, y2_ref, dinv_ref, b2_ref, o_ref):
    logits = (dinv_ref[...] * (acc_ref[0] + acc_ref[1] + y2_ref[...])
              + b2_ref[...])
    m = jnp.max(logits, axis=1, keepdims=True)
    z = logits - m
    lse = jnp.log(jnp.sum(jnp.exp(z), axis=1, keepdims=True))
    o_ref[...] = z - lse


def _final(acc2, y2, dinv, b2):
    return pl.pallas_call(
        _final_body,
        grid=(GRID,),
        in_specs=[
            pl.BlockSpec((NC, BLK, N_CLASSES), lambda i: (0, i, 0)),
            pl.BlockSpec((BLK, N_CLASSES), lambda i: (i, 0)),
            pl.BlockSpec((BLK, 1), lambda i: (i, 0)),
            pl.BlockSpec((1, N_CLASSES), lambda i: (0, 0)),
        ],
        out_specs=pl.BlockSpec((BLK, N_CLASSES), lambda i: (i, 0)),
        out_shape=jax.ShapeDtypeStruct((N_NODES, N_CLASSES), jnp.float32),
    )(acc2, y2, dinv, b2)


def kernel(x, edge_index, W1, b1, W2, b2):
    ei = edge_index.astype(jnp.int32)
    src32 = ei[0].reshape(NW, NCH, CHUNK)
    dst32 = ei[1].reshape(NW, NCH, CHUNK)
    src16 = ei[0].reshape(NS, N_EDGES // NS // CHUNK, CHUNK)
    dst16 = ei[1].reshape(NS, N_EDGES // NS // CHUNK, CHUNK)

    hist = _degree_kernel(dst32).reshape(NW, N_NODES)
    xw1 = _mm1(x, W1)
    dinv = _dinv(hist)
    y1 = _scale(xw1, dinv)
    acc1 = _prop_cols(y1, src16, dst16)
    y2 = _mid(acc1, y1, dinv, b1.reshape(1, HIDDEN), W2)
    acc2 = _prop_edges(y2, src32, dst32)
    return _final(acc2, y2, dinv, b2.reshape(1, N_CLASSES))


# speedup vs baseline: 34.0565x; 1.8613x over previous
import functools

import jax
import jax.numpy as jnp
from jax import lax
from jax.experimental import pallas as pl
from jax.experimental.pallas import tpu as pltpu
from jax.experimental.pallas import tpu_sc as plsc

N_NODES = 10000
N_EDGES = 320000
D_FEAT = 128
HIDDEN = 128
N_CLASSES = 64

NC = 2
NS = 16
NW = NC * NS
EPT = N_EDGES // NW
CHUNK = 80
NCH = EPT // CHUNK
NBUF = 5
N_PAD = 10240
RPT = N_PAD // NS
ZROWS = 128

BLK = 1000
GRID = N_NODES // BLK

_mesh = lambda: plsc.VectorSubcoreMesh(core_axis_name="c", subcore_axis_name="s")
_SC_PARAMS = pltpu.CompilerParams(needs_layout_passes=False,
                                  use_tc_tiling_on_sc=False)


@functools.partial(
    pl.kernel,
    out_type=jax.ShapeDtypeStruct((NW, 1, N_NODES), jnp.float32),
    mesh=_mesh(),
    compiler_params=_SC_PARAMS,
    scratch_types=[
        pltpu.VMEM((NCH, CHUNK), jnp.int32),
        pltpu.VMEM((N_NODES,), jnp.float32),
    ],
)
def _degree_kernel(dst_hbm, out_hbm, idx_v, hist_v):
    c = lax.axis_index("c")
    s = lax.axis_index("s")
    wid = s * NC + c
    pltpu.sync_copy(dst_hbm.at[wid], idx_v)
    zeros = jnp.zeros((16,), jnp.float32)

    def zbody(i, carry):
        hist_v[pl.ds(i * 16, 16)] = zeros
        return carry

    lax.fori_loop(0, N_NODES // 16, zbody, 0)
    ones = jnp.ones((16,), jnp.float32)

    def body(j, carry):
        for l in range(CHUNK // 16):
            idx16 = idx_v[j, pl.ds(l * 16, 16)]
            plsc.addupdate_scatter(hist_v, [idx16], ones)
        return carry

    lax.fori_loop(0, NCH, body, 0)
    pltpu.sync_copy(hist_v, out_hbm.at[wid, 0])


def _make_propagate(mode):
    nch = N_EDGES // NS // CHUNK if mode == "cols" else NCH

    ngroups = nch // NBUF

    @functools.partial(
        pl.kernel,
        out_type=jax.ShapeDtypeStruct((NC, N_PAD, 64), jnp.float32),
        mesh=_mesh(),
        compiler_params=_SC_PARAMS,
        scratch_types=[
            pltpu.VMEM((nch, CHUNK), jnp.int32),
            pltpu.VMEM((nch, CHUNK), jnp.int32),
            [pltpu.VMEM((CHUNK, 64), jnp.float32) for _ in range(NBUF)],
            pltpu.VMEM((ZROWS, 64), jnp.float32),
            pltpu.VMEM_SHARED((N_PAD, 64), jnp.float32),
            [pltpu.SemaphoreType.DMA for _ in range(NBUF)],
            [pltpu.SemaphoreType.DMA for _ in range(NBUF)],
        ],
    )
    def prop(y_hbm, src_hbm, dst_hbm, out_hbm, src_v, dst_v, rows, zbuf, acc,
             gsem, ssem):
        c = lax.axis_index("c")
        s = lax.axis_index("s")
        if mode == "cols":
            widx = s
            ysrc = y_hbm.at[c]
        else:
            widx = s * NC + c
            ysrc = y_hbm
        pltpu.sync_copy(src_hbm.at[widx], src_v)
        pltpu.sync_copy(dst_hbm.at[widx], dst_v)

        zeros = jnp.zeros((16,), jnp.float32)

        def zbody(i, carry):
            for l in range(4):
                zbuf[i, pl.ds(l * 16, 16)] = zeros
            return carry

        lax.fori_loop(0, ZROWS, zbody, 0)
        for k in range(RPT // ZROWS):
            pltpu.sync_copy(zbuf, acc.at[pl.ds(s * RPT + k * ZROWS, ZROWS)])
        plsc.subcore_barrier()

        for b in range(NBUF):
            pltpu.async_copy(ysrc.at[src_v.at[b]], rows[b], gsem[b])

        def body(g, carry):
            for b in range(NBUF):
                j = g * NBUF + b
                pltpu.make_async_copy(ysrc.at[src_v.at[j]], rows[b],
                                      gsem[b]).wait()
                pltpu.async_copy(rows[b], acc.at[dst_v.at[j]], ssem[b],
                                 add=True)
            for b in range(NBUF):
                j = g * NBUF + b
                pltpu.make_async_copy(rows[b], acc.at[dst_v.at[j]],
                                      ssem[b]).wait()

                @pl.when(g + 1 < ngroups)
                def _():
                    pltpu.async_copy(ysrc.at[src_v.at[j + NBUF]], rows[b],
                                     gsem[b])

            return carry

        lax.fori_loop(0, ngroups, body, 0)
        plsc.subcore_barrier()
        pltpu.sync_copy(acc.at[pl.ds(s * RPT, RPT)],
                        out_hbm.at[c].at[pl.ds(s * RPT, RPT)])

    return prop


_prop_cols = _make_propagate("cols")
_prop_edges = _make_propagate("edges")


def _mm1_body(x_ref, w_ref, o_ref):
    o_ref[...] = jnp.dot(x_ref[...], w_ref[...],
                         preferred_element_type=jnp.float32)


def _mm1(x, w):
    return pl.pallas_call(
        _mm1_body,
        grid=(GRID,),
        in_specs=[
            pl.BlockSpec((BLK, D_FEAT), lambda i: (i, 0)),
            pl.BlockSpec((D_FEAT, HIDDEN), lambda i: (0, 0)),
        ],
        out_specs=pl.BlockSpec((BLK, HIDDEN), lambda i: (i, 0)),
        out_shape=jax.ShapeDtypeStruct((N_NODES, HIDDEN), jnp.float32),
    )(x, w)


def _dinv_body(hist_ref, dinv_ref):
    ones = jnp.ones((NW, 1), jnp.float32)
    deg = lax.dot_general(hist_ref[...], ones, (((0,), (0,)), ((), ())),
                          preferred_element_type=jnp.float32) + 1.0
    dinv_ref[...] = lax.rsqrt(deg)


def _dinv(hist):
    return pl.pallas_call(
        _dinv_body,
        out_shape=jax.ShapeDtypeStruct((N_NODES, 1), jnp.float32),
    )(hist)


def _scale_body(xw_ref, dinv_ref, y_ref):
    y = xw_ref[...] * dinv_ref[...]
    y_ref[0, ...] = y[:, :64]
    y_ref[1, ...] = y[:, 64:]


def _scale(xw, dinv):
    return pl.pallas_call(
        _scale_body,
        grid=(GRID,),
        in_specs=[
            pl.BlockSpec((BLK, HIDDEN), lambda i: (i, 0)),
            pl.BlockSpec((BLK, 1), lambda i: (i, 0)),
        ],
        out_specs=pl.BlockSpec((NC, BLK, 64), lambda i: (0, i, 0)),
        out_shape=jax.ShapeDtypeStruct((NC, N_NODES, 64), jnp.float32),
    )(xw, dinv)


def _mid_body(acc_ref, y1_ref, dinv_ref, b1_ref, w2_ref, y2_ref):
    dinv = dinv_ref[...]
    prop1 = jnp.concatenate([acc_ref[0] + y1_ref[0], acc_ref[1] + y1_ref[1]],
                            axis=1)
    h = jnp.maximum(dinv * prop1 + b1_ref[...], 0.0)
    y2_ref[...] = jnp.dot(h, w2_ref[...],
                          preferred_element_type=jnp.float32) * dinv


def _mid(acc1, y1, dinv, b1, w2):
    return pl.pallas_call(
        _mid_body,
        grid=(GRID,),
        in_specs=[
            pl.BlockSpec((NC, BLK, 64), lambda i: (0, i, 0)),
            pl.BlockSpec((NC, BLK, 64), lambda i: (0, i, 0)),
            pl.BlockSpec((BLK, 1), lambda i: (i, 0)),
            pl.BlockSpec((1, HIDDEN), lambda i: (0, 0)),
            pl.BlockSpec((HIDDEN, N_CLASSES), lambda i: (0, 0)),
        ],
        out_specs=pl.BlockSpec((BLK, N_CLASSES), lambda i: (i, 0)),
        out_shape=jax.ShapeDtypeStruct((N_NODES, N_CLASSES), jnp.float32),
    )(acc1, y1, dinv, b1, w2)


def _final_body(acc_ref, y2_ref, dinv_ref, b2_ref, o_ref):
    logits = (dinv_ref[...] * (acc_ref[0] + acc_ref[1] + y2_ref[...])
              + b2_ref[...])
    m = jnp.max(logits, axis=1, keepdims=True)
    z = logits - m
    lse = jnp.log(jnp.sum(jnp.exp(z), axis=1, keepdims=True))
    o_ref[...] = z - lse


def _final(acc2, y2, dinv, b2):
    return pl.pallas_call(
        _final_body,
        grid=(GRID,),
        in_specs=[
            pl.BlockSpec((NC, BLK, N_CLASSES), lambda i: (0, i, 0)),
            pl.BlockSpec((BLK, N_CLASSES), lambda i: (i, 0)),
            pl.BlockSpec((BLK, 1), lambda i: (i, 0)),
            pl.BlockSpec((1, N_CLASSES), lambda i: (0, 0)),
        ],
        out_specs=pl.BlockSpec((BLK, N_CLASSES), lambda i: (i, 0)),
        out_shape=jax.ShapeDtypeStruct((N_NODES, N_CLASSES), jnp.float32),
    )(acc2, y2, dinv, b2)


def kernel(x, edge_index, W1, b1, W2, b2):
    ei = edge_index.astype(jnp.int32)
    src32 = ei[0].reshape(NW, NCH, CHUNK)
    dst32 = ei[1].reshape(NW, NCH, CHUNK)
    src16 = ei[0].reshape(NS, N_EDGES // NS // CHUNK, CHUNK)
    dst16 = ei[1].reshape(NS, N_EDGES // NS // CHUNK, CHUNK)

    hist = _degree_kernel(dst32).reshape(NW, N_NODES)
    xw1 = _mm1(x, W1)
    dinv = _dinv(hist)
    y1 = _scale(xw1, dinv)
    acc1 = _prop_cols(y1, src16, dst16)
    y2 = _mid(acc1, y1, dinv, b1.reshape(1, HIDDEN), W2)
    acc2 = _prop_edges(y2, src32, dst32)
    return _final(acc2, y2, dinv, b2.reshape(1, N_CLASSES))


# speedup vs baseline: 34.9226x; 1.0254x over previous
import functools

import jax
import jax.numpy as jnp
from jax import lax
from jax.experimental import pallas as pl
from jax.experimental.pallas import tpu as pltpu
from jax.experimental.pallas import tpu_sc as plsc

N_NODES = 10000
N_EDGES = 320000
D_FEAT = 128
HIDDEN = 128
N_CLASSES = 64

NC = 2
NS = 16
NW = NC * NS
EPT = N_EDGES // NW
CHUNK = 80
NCH = EPT // CHUNK
NBUF = 5
N_PAD = 10240
RPT = N_PAD // NS
ZROWS = 128

BLK = 1000
GRID = N_NODES // BLK

_mesh = lambda: plsc.VectorSubcoreMesh(core_axis_name="c", subcore_axis_name="s")
_SC_PARAMS = pltpu.CompilerParams(needs_layout_passes=False,
                                  use_tc_tiling_on_sc=False)


@functools.partial(
    pl.kernel,
    out_type=jax.ShapeDtypeStruct((NW, 1, N_NODES), jnp.float32),
    mesh=_mesh(),
    compiler_params=_SC_PARAMS,
    scratch_types=[
        pltpu.VMEM((NCH, CHUNK), jnp.int32),
        pltpu.VMEM((N_NODES,), jnp.float32),
    ],
)
def _degree_kernel(dst_hbm, out_hbm, idx_v, hist_v):
    c = lax.axis_index("c")
    s = lax.axis_index("s")
    wid = s * NC + c
    pltpu.sync_copy(dst_hbm.at[wid], idx_v)
    zeros = jnp.zeros((16,), jnp.float32)

    def zbody(i, carry):
        hist_v[pl.ds(i * 16, 16)] = zeros
        return carry

    lax.fori_loop(0, N_NODES // 16, zbody, 0)
    ones = jnp.ones((16,), jnp.float32)

    def body(j, carry):
        for l in range(CHUNK // 16):
            idx16 = idx_v[j, pl.ds(l * 16, 16)]
            plsc.addupdate_scatter(hist_v, [idx16], ones)
        return carry

    lax.fori_loop(0, NCH, body, 0)
    pltpu.sync_copy(hist_v, out_hbm.at[wid, 0])


def _make_propagate(mode):
    nch = N_EDGES // NS // CHUNK if mode == "cols" else NCH

    ngroups = nch // NBUF

    @functools.partial(
        pl.kernel,
        out_type=jax.ShapeDtypeStruct((NC, N_PAD, 64), jnp.float32),
        mesh=_mesh(),
        compiler_params=_SC_PARAMS,
        scratch_types=[
            pltpu.VMEM((nch, CHUNK), jnp.int32),
            pltpu.VMEM((nch, CHUNK), jnp.int32),
            [pltpu.VMEM((CHUNK, 64), jnp.float32) for _ in range(NBUF)],
            pltpu.VMEM((ZROWS, 64), jnp.float32),
            pltpu.VMEM_SHARED((N_PAD, 64), jnp.float32),
            [pltpu.SemaphoreType.DMA for _ in range(NBUF)],
            [pltpu.SemaphoreType.DMA for _ in range(NBUF)],
        ],
    )
    def prop(y_hbm, src_hbm, dst_hbm, out_hbm, src_v, dst_v, rows, zbuf, acc,
             gsem, ssem):
        c = lax.axis_index("c")
        s = lax.axis_index("s")
        if mode == "cols":
            widx = s
            ysrc = y_hbm.at[c]
        else:
            widx = s * NC + c
            ysrc = y_hbm
        pltpu.sync_copy(src_hbm.at[widx], src_v)
        pltpu.sync_copy(dst_hbm.at[widx], dst_v)

        zeros = jnp.zeros((16,), jnp.float32)

        def zbody(i, carry):
            for l in range(4):
                zbuf[i, pl.ds(l * 16, 16)] = zeros
            return carry

        lax.fori_loop(0, ZROWS, zbody, 0)
        for k in range(RPT // ZROWS):
            pltpu.sync_copy(zbuf, acc.at[pl.ds(s * RPT + k * ZROWS, ZROWS)])
        plsc.subcore_barrier()

        for b in range(NBUF):
            pltpu.async_copy(ysrc.at[src_v.at[b]], rows[b], gsem[b])

        def body(g, carry):
            for b in range(NBUF):
                j = g * NBUF + b
                pltpu.make_async_copy(ysrc.at[src_v.at[j]], rows[b],
                                      gsem[b]).wait()
                pltpu.async_copy(rows[b], acc.at[dst_v.at[j]], ssem[b],
                                 add=True)
            for b in range(NBUF):
                j = g * NBUF + b
                pltpu.make_async_copy(rows[b], acc.at[dst_v.at[j]],
                                      ssem[b]).wait()

                @pl.when(g + 1 < ngroups)
                def _():
                    pltpu.async_copy(ysrc.at[src_v.at[j + NBUF]], rows[b],
                                     gsem[b])

            return carry

        lax.fori_loop(0, ngroups, body, 0)
        plsc.subcore_barrier()
        pltpu.sync_copy(acc.at[pl.ds(s * RPT, RPT)],
                        out_hbm.at[c].at[pl.ds(s * RPT, RPT)])

    return prop


_prop_cols = _make_propagate("cols")
_prop_edges = _make_propagate("edges")


def _front_body(hist_ref, x_ref, w_ref, y_ref, dinv_ref):
    ones = jnp.ones((NW, 1), jnp.float32)
    deg = lax.dot_general(hist_ref[...], ones, (((0,), (0,)), ((), ())),
                          preferred_element_type=jnp.float32) + 1.0
    dinv = lax.rsqrt(deg)
    dinv_ref[...] = dinv
    y = jnp.dot(x_ref[...], w_ref[...],
                preferred_element_type=jnp.float32) * dinv
    y_ref[0, ...] = y[:, :64]
    y_ref[1, ...] = y[:, 64:]


def _front(hist, x, w):
    return pl.pallas_call(
        _front_body,
        out_shape=[
            jax.ShapeDtypeStruct((NC, N_NODES, 64), jnp.float32),
            jax.ShapeDtypeStruct((N_NODES, 1), jnp.float32),
        ],
    )(hist, x, w)


def _mid_body(acc_ref, y1_ref, dinv_ref, b1_ref, w2_ref, y2_ref):
    dinv = dinv_ref[...]
    prop1 = jnp.concatenate([acc_ref[0] + y1_ref[0], acc_ref[1] + y1_ref[1]],
                            axis=1)
    h = jnp.maximum(dinv * prop1 + b1_ref[...], 0.0)
    y2_ref[...] = jnp.dot(h, w2_ref[...],
                          preferred_element_type=jnp.float32) * dinv


def _mid(acc1, y1, dinv, b1, w2):
    return pl.pallas_call(
        _mid_body,
        grid=(GRID,),
        in_specs=[
            pl.BlockSpec((NC, BLK, 64), lambda i: (0, i, 0)),
            pl.BlockSpec((NC, BLK, 64), lambda i: (0, i, 0)),
            pl.BlockSpec((BLK, 1), lambda i: (i, 0)),
            pl.BlockSpec((1, HIDDEN), lambda i: (0, 0)),
            pl.BlockSpec((HIDDEN, N_CLASSES), lambda i: (0, 0)),
        ],
        out_specs=pl.BlockSpec((BLK, N_CLASSES), lambda i: (i, 0)),
        out_shape=jax.ShapeDtypeStruct((N_NODES, N_CLASSES), jnp.float32),
    )(acc1, y1, dinv, b1, w2)


def _final_body(acc_ref, y2_ref, dinv_ref, b2_ref, o_ref):
    logits = (dinv_ref[...] * (acc_ref[0] + acc_ref[1] + y2_ref[...])
              + b2_ref[...])
    m = jnp.max(logits, axis=1, keepdims=True)
    z = logits - m
    lse = jnp.log(jnp.sum(jnp.exp(z), axis=1, keepdims=True))
    o_ref[...] = z - lse


def _final(acc2, y2, dinv, b2):
    return pl.pallas_call(
        _final_body,
        grid=(GRID,),
        in_specs=[
            pl.BlockSpec((NC, BLK, N_CLASSES), lambda i: (0, i, 0)),
            pl.BlockSpec((BLK, N_CLASSES), lambda i: (i, 0)),
            pl.BlockSpec((BLK, 1), lambda i: (i, 0)),
            pl.BlockSpec((1, N_CLASSES), lambda i: (0, 0)),
        ],
        out_specs=pl.BlockSpec((BLK, N_CLASSES), lambda i: (i, 0)),
        out_shape=jax.ShapeDtypeStruct((N_NODES, N_CLASSES), jnp.float32),
    )(acc2, y2, dinv, b2)


def kernel(x, edge_index, W1, b1, W2, b2):
    ei = edge_index.astype(jnp.int32)
    src32 = ei[0].reshape(NW, NCH, CHUNK)
    dst32 = ei[1].reshape(NW, NCH, CHUNK)
    src16 = ei[0].reshape(NS, N_EDGES // NS // CHUNK, CHUNK)
    dst16 = ei[1].reshape(NS, N_EDGES // NS // CHUNK, CHUNK)

    hist = _degree_kernel(dst32).reshape(NW, N_NODES)
    y1, dinv = _front(hist, x, W1)
    acc1 = _prop_cols(y1, src16, dst16)
    y2 = _mid(acc1, y1, dinv, b1.reshape(1, HIDDEN), W2)
    acc2 = _prop_edges(y2, src32, dst32)
    return _final(acc2, y2, dinv, b2.reshape(1, N_CLASSES))
